# initial kernel scaffold (unmeasured)
import jax
import jax.numpy as jnp
from jax import lax
from jax.experimental import pallas as pl
from jax.experimental.pallas import tpu as pltpu

N_DEV = 4
M, K, N = 4096, 4096, 8192
CHUNK = M // N_DEV
SUB = 256
N_SUB = CHUNK // SUB


def _ar_body(p_ref, out_ref, recv_hbm, a_vmem, b_vmem,
             rs_send, rs_recv, ag_send, ag_recv, copy_sems):
    my = lax.axis_index("i")
    right = lax.rem(my + 1, N_DEV)

    for s in range(N_DEV - 1):
        c_send = lax.rem(my - s + N_DEV, N_DEV)
        c_recv = lax.rem(my - s - 1 + N_DEV, N_DEV)
        src = p_ref if s == 0 else out_ref
        rdma = pltpu.make_async_remote_copy(
            src_ref=src.at[pl.ds(c_send * CHUNK, CHUNK)],
            dst_ref=recv_hbm.at[s],
            send_sem=rs_send.at[s],
            recv_sem=rs_recv.at[s],
            device_id=(right,),
            device_id_type=pl.DeviceIdType.MESH,
        )
        rdma.start()
        rdma.wait()
        for t in range(N_SUB):
            row = c_recv * CHUNK + t * SUB
            cp_a = pltpu.make_async_copy(
                recv_hbm.at[s, pl.ds(t * SUB, SUB)], a_vmem, copy_sems.at[0])
            cp_b = pltpu.make_async_copy(
                p_ref.at[pl.ds(row, SUB)], b_vmem, copy_sems.at[1])
            cp_a.start()
            cp_b.start()
            cp_a.wait()
            cp_b.wait()
            a_vmem[...] = a_vmem[...] + b_vmem[...]
            cp_o = pltpu.make_async_copy(
                a_vmem, out_ref.at[pl.ds(row, SUB)], copy_sems.at[2])
            cp_o.start()
            cp_o.wait()

    for s in range(N_DEV - 1):
        c = lax.rem(my + 1 - s + N_DEV, N_DEV)
        rdma = pltpu.make_async_remote_copy(
            src_ref=out_ref.at[pl.ds(c * CHUNK, CHUNK)],
            dst_ref=out_ref.at[pl.ds(c * CHUNK, CHUNK)],
            send_sem=ag_send.at[s],
            recv_sem=ag_recv.at[s],
            device_id=(right,),
            device_id_type=pl.DeviceIdType.MESH,
        )
        rdma.start()
        rdma.wait()


def _all_reduce(partial):
    return pl.pallas_call(
        _ar_body,
        out_shape=jax.ShapeDtypeStruct((M, N), jnp.float32),
        in_specs=[pl.BlockSpec(memory_space=pl.ANY)],
        out_specs=pl.BlockSpec(memory_space=pl.ANY),
        scratch_shapes=[
            pltpu.HBM((N_DEV - 1, CHUNK, N), jnp.float32),
            pltpu.VMEM((SUB, N), jnp.float32),
            pltpu.VMEM((SUB, N), jnp.float32),
            pltpu.SemaphoreType.DMA((N_DEV - 1,)),
            pltpu.SemaphoreType.DMA((N_DEV - 1,)),
            pltpu.SemaphoreType.DMA((N_DEV - 1,)),
            pltpu.SemaphoreType.DMA((N_DEV - 1,)),
            pltpu.SemaphoreType.DMA((3,)),
        ],
    )(partial)


def kernel(x, w_mat):
    partial = jnp.dot(x, w_mat, precision=lax.Precision.HIGHEST)
    red = _all_reduce(partial)
    y = jnp.maximum(red, 0.0)
    scale = jnp.max(y) / 448.0
    q = (y / scale).astype(jnp.float8_e4m3fn)
    return q.astype(jnp.float32) * scale


# baseline (device time: 2895608 ns/iter reference)
import jax
import jax.numpy as jnp
from jax import lax
from jax.experimental import pallas as pl
from jax.experimental.pallas import tpu as pltpu

N_DEV = 4
M, K, N = 4096, 4096, 8192
CHUNK = M // N_DEV
SUB = 256
N_SUB = CHUNK // SUB


def _ar_body(p_ref, out_ref, recv_hbm, a_vmem, b_vmem,
             rs_send, rs_recv, ag_send, ag_recv, copy_sems):
    my = lax.axis_index("i")
    right = lax.rem(my + 1, N_DEV)

    for s in range(N_DEV - 1):
        c_send = lax.rem(my - s + N_DEV, N_DEV)
        c_recv = lax.rem(my - s - 1 + N_DEV, N_DEV)
        src = p_ref if s == 0 else out_ref
        rdma = pltpu.make_async_remote_copy(
            src_ref=src.at[pl.ds(c_send * CHUNK, CHUNK)],
            dst_ref=recv_hbm.at[s],
            send_sem=rs_send.at[s],
            recv_sem=rs_recv.at[s],
            device_id=(right,),
            device_id_type=pl.DeviceIdType.MESH,
        )
        rdma.start()
        rdma.wait()
        for t in range(N_SUB):
            row = c_recv * CHUNK + t * SUB
            cp_a = pltpu.make_async_copy(
                recv_hbm.at[s, pl.ds(t * SUB, SUB)], a_vmem, copy_sems.at[0])
            cp_b = pltpu.make_async_copy(
                p_ref.at[pl.ds(row, SUB)], b_vmem, copy_sems.at[1])
            cp_a.start()
            cp_b.start()
            cp_a.wait()
            cp_b.wait()
            a_vmem[...] = a_vmem[...] + b_vmem[...]
            cp_o = pltpu.make_async_copy(
                a_vmem, out_ref.at[pl.ds(row, SUB)], copy_sems.at[2])
            cp_o.start()
            cp_o.wait()

    for s in range(N_DEV - 1):
        c = lax.rem(my + 1 - s + N_DEV, N_DEV)
        rdma = pltpu.make_async_remote_copy(
            src_ref=out_ref.at[pl.ds(c * CHUNK, CHUNK)],
            dst_ref=out_ref.at[pl.ds(c * CHUNK, CHUNK)],
            send_sem=ag_send.at[s],
            recv_sem=ag_recv.at[s],
            device_id=(right,),
            device_id_type=pl.DeviceIdType.MESH,
        )
        rdma.start()
        rdma.wait()


def _all_reduce(partial):
    out, _ = pl.pallas_call(
        _ar_body,
        out_shape=[
            jax.ShapeDtypeStruct((M, N), jnp.float32),
            jax.ShapeDtypeStruct((N_DEV - 1, CHUNK, N), jnp.float32),
        ],
        in_specs=[pl.BlockSpec(memory_space=pl.ANY)],
        out_specs=[
            pl.BlockSpec(memory_space=pl.ANY),
            pl.BlockSpec(memory_space=pl.ANY),
        ],
        scratch_shapes=[
            pltpu.VMEM((SUB, N), jnp.float32),
            pltpu.VMEM((SUB, N), jnp.float32),
            pltpu.SemaphoreType.DMA((N_DEV - 1,)),
            pltpu.SemaphoreType.DMA((N_DEV - 1,)),
            pltpu.SemaphoreType.DMA((N_DEV - 1,)),
            pltpu.SemaphoreType.DMA((N_DEV - 1,)),
            pltpu.SemaphoreType.DMA((3,)),
        ],
    )(partial)
    return out


def kernel(x, w_mat):
    partial = jnp.dot(x, w_mat, precision=lax.Precision.HIGHEST)
    red = _all_reduce(partial)
    y = jnp.maximum(red, 0.0)
    scale = jnp.max(y) / 448.0
    q = (y / scale).astype(jnp.float8_e4m3fn)
    q = lax.optimization_barrier(q)
    return q.astype(jnp.float32) * scale


# device time: 1464883 ns/iter; 1.9767x vs baseline; 1.9767x over previous
import jax
import jax.numpy as jnp
from jax import lax
from jax.experimental import pallas as pl
from jax.experimental.pallas import tpu as pltpu

N_DEV = 4
M, K, N = 4096, 4096, 8192
CHUNK = M // N_DEV
HALF = N // 2
SUB = 512
N_SUB = CHUNK // SUB


def _ar_body(p_ref, out_ref, out8, recv_hbm,
             a_vmem, b_vmem, q8_vmem, mx_comb, mx_recv,
             rs_send, rs_recv, mx_send, mx_recv_sems, ag_send, ag_recv,
             copy_sems):
    my = lax.axis_index("i")
    right = lax.rem(my + 1, N_DEV)
    left = lax.rem(my + N_DEV - 1, N_DEV)

    def rows(c):
        return pl.ds(c * CHUNK, CHUNK)

    cols = (pl.ds(0, HALF), pl.ds(HALF, HALF))

    amax = jnp.float32(0.0)

    for s in range(N_DEV - 1):
        rdmas = []
        for r in range(2):
            if r == 0:
                c_send = lax.rem(my - s + N_DEV, N_DEV)
                dst_dev = right
            else:
                c_send = lax.rem(my + s, N_DEV)
                dst_dev = left
            src = p_ref if s == 0 else out_ref
            rdma = pltpu.make_async_remote_copy(
                src_ref=src.at[rows(c_send), cols[r]],
                dst_ref=recv_hbm.at[r, s],
                send_sem=rs_send.at[r, s],
                recv_sem=rs_recv.at[r, s],
                device_id=(dst_dev,),
                device_id_type=pl.DeviceIdType.MESH,
            )
            rdma.start()
            rdmas.append(rdma)
        for rdma in rdmas:
            rdma.wait()
        for r in range(2):
            if r == 0:
                c_recv = lax.rem(my - s - 1 + N_DEV, N_DEV)
            else:
                c_recv = lax.rem(my + s + 1, N_DEV)
            for t in range(N_SUB):
                tr = pl.ds(t * SUB, SUB)
                trg = pl.ds(c_recv * CHUNK + t * SUB, SUB)
                cp_a = pltpu.make_async_copy(recv_hbm.at[r, s, tr], a_vmem,
                                             copy_sems.at[0])
                cp_b = pltpu.make_async_copy(p_ref.at[trg, cols[r]],
                                             b_vmem, copy_sems.at[1])
                cp_a.start()
                cp_b.start()
                cp_a.wait()
                cp_b.wait()
                a_vmem[...] = a_vmem[...] + b_vmem[...]
                if s == N_DEV - 2:
                    amax = jnp.maximum(amax, jnp.max(a_vmem[...]))
                cp_o = pltpu.make_async_copy(a_vmem, out_ref.at[trg, cols[r]],
                                             copy_sems.at[2])
                cp_o.start()
                cp_o.wait()

    mx_comb[...] = jnp.full((8, 128), jnp.maximum(amax, 0.0), jnp.float32)
    for h in range(N_DEV - 1):
        rdma = pltpu.make_async_remote_copy(
            src_ref=mx_comb,
            dst_ref=mx_recv.at[h],
            send_sem=mx_send.at[h],
            recv_sem=mx_recv_sems.at[h],
            device_id=(right,),
            device_id_type=pl.DeviceIdType.MESH,
        )
        rdma.start()
        rdma.wait()
        mx_comb[...] = jnp.maximum(mx_comb[...], mx_recv[h])
    scale = jnp.max(mx_comb[...]) / 448.0

    own = (lax.rem(my + 1, N_DEV), lax.rem(my + N_DEV - 1, N_DEV))
    for r in range(2):
        for t in range(N_SUB):
            trg = pl.ds(own[r] * CHUNK + t * SUB, SUB)
            cp = pltpu.make_async_copy(out_ref.at[trg, cols[r]], a_vmem,
                                       copy_sems.at[0])
            cp.start()
            cp.wait()
            q8_vmem[...] = (jnp.maximum(a_vmem[...], 0.0) / scale).astype(
                jnp.float8_e4m3fn)
            cp = pltpu.make_async_copy(q8_vmem, out8.at[trg, cols[r]],
                                       copy_sems.at[1])
            cp.start()
            cp.wait()

    for s in range(N_DEV - 1):
        rdmas = []
        for r in range(2):
            if r == 0:
                c = lax.rem(my + 1 - s + N_DEV, N_DEV)
                dst_dev = right
            else:
                c = lax.rem(my - 1 + s + N_DEV, N_DEV)
                dst_dev = left
            rdma = pltpu.make_async_remote_copy(
                src_ref=out8.at[rows(c), cols[r]],
                dst_ref=out8.at[rows(c), cols[r]],
                send_sem=ag_send.at[r, s],
                recv_sem=ag_recv.at[r, s],
                device_id=(dst_dev,),
                device_id_type=pl.DeviceIdType.MESH,
            )
            rdma.start()
            rdmas.append(rdma)
        for rdma in rdmas:
            rdma.wait()

    for c in range(N_DEV):
        for r in range(2):
            for t in range(N_SUB):
                trg = pl.ds(c * CHUNK + t * SUB, SUB)
                cp = pltpu.make_async_copy(out8.at[trg, cols[r]], q8_vmem,
                                           copy_sems.at[0])
                cp.start()
                cp.wait()
                a_vmem[...] = q8_vmem[...].astype(jnp.float32) * scale
                cp = pltpu.make_async_copy(a_vmem, out_ref.at[trg, cols[r]],
                                           copy_sems.at[1])
                cp.start()
                cp.wait()


def _all_reduce(partial):
    out, _, _ = pl.pallas_call(
        _ar_body,
        out_shape=[
            jax.ShapeDtypeStruct((M, N), jnp.float32),
            jax.ShapeDtypeStruct((M, N), jnp.float8_e4m3fn),
            jax.ShapeDtypeStruct((2, N_DEV - 1, CHUNK, HALF), jnp.float32),
        ],
        in_specs=[pl.BlockSpec(memory_space=pl.ANY)],
        out_specs=[
            pl.BlockSpec(memory_space=pl.ANY),
            pl.BlockSpec(memory_space=pl.ANY),
            pl.BlockSpec(memory_space=pl.ANY),
        ],
        scratch_shapes=[
            pltpu.VMEM((SUB, HALF), jnp.float32),
            pltpu.VMEM((SUB, HALF), jnp.float32),
            pltpu.VMEM((SUB, HALF), jnp.float8_e4m3fn),
            pltpu.VMEM((8, 128), jnp.float32),
            pltpu.VMEM((N_DEV - 1, 8, 128), jnp.float32),
            pltpu.SemaphoreType.DMA((2, N_DEV - 1)),
            pltpu.SemaphoreType.DMA((2, N_DEV - 1)),
            pltpu.SemaphoreType.DMA((N_DEV - 1,)),
            pltpu.SemaphoreType.DMA((N_DEV - 1,)),
            pltpu.SemaphoreType.DMA((2, N_DEV - 1)),
            pltpu.SemaphoreType.DMA((2, N_DEV - 1)),
            pltpu.SemaphoreType.DMA((3,)),
        ],
    )(partial)
    return out


def kernel(x, w_mat):
    partial = jnp.dot(x, w_mat, precision=lax.Precision.HIGHEST)
    return _all_reduce(partial)


# device time: 1072581 ns/iter; 2.6997x vs baseline; 1.3658x over previous
import os as _os

import jax

jax.config.update("jax_compilation_cache_dir",
                  _os.path.join(_os.path.dirname(__file__), ".jax_cache"))
jax.config.update("jax_persistent_cache_min_compile_time_secs", 0)
jax.config.update("jax_persistent_cache_min_entry_size_bytes", 0)

import jax.numpy as jnp
from jax import lax
from jax.experimental import pallas as pl
from jax.experimental.pallas import tpu as pltpu

N_DEV = 4
M, K, N = 4096, 4096, 8192
KSH = K // N_DEV
CHUNK = M // N_DEV
HALF = N // 2
SUB = 256
N_SUB = CHUNK // SUB
GR = 512
GC = 2048


def _ar_body(x_ref, w_ref, out_ref, out8, part, recv_hbm,
             a_vmem, b_vmem, q8_vmem, mx_comb, mx_recv,
             xf_vmem, x_hi, x_lo, w_hi, w_lo, g_vmem,
             rs_send, rs_recv, mx_send, mx_recv_sems, ag_send, ag_recv,
             copy_sems, gx_sems, gw_sems, gg_sems):
    my = lax.axis_index("i")
    right = lax.rem(my + 1, N_DEV)
    left = lax.rem(my + N_DEV - 1, N_DEV)

    def rows(c):
        return pl.ds(c * CHUNK, CHUNK)

    cols = (pl.ds(0, HALF), pl.ds(HALF, HALF))

    def gemm_piece(c, r):
        base = r * HALF
        cp = pltpu.make_async_copy(x_ref.at[rows(c)], xf_vmem, gx_sems.at[0])
        cp.start()
        cp.wait()
        x_hi[...] = xf_vmem[...].astype(jnp.bfloat16)
        x_lo[...] = (xf_vmem[...] - x_hi[...].astype(jnp.float32)).astype(
            jnp.bfloat16)
        for j in range(2):
            bcols = pl.ds(base + j * GC, GC)
            cp = pltpu.make_async_copy(w_ref.at[:, bcols], g_vmem,
                                       gw_sems.at[0])
            cp.start()
            cp.wait()
            w_hi[...] = g_vmem[...].astype(jnp.bfloat16)
            w_lo[...] = (g_vmem[...] - w_hi[...].astype(jnp.float32)).astype(
                jnp.bfloat16)
            g_vmem[...] = jnp.dot(x_hi[...], w_hi[...],
                                  preferred_element_type=jnp.float32)
            g_vmem[...] = g_vmem[...] + jnp.dot(
                x_lo[...], w_hi[...], preferred_element_type=jnp.float32)
            g_vmem[...] = g_vmem[...] + jnp.dot(
                x_hi[...], w_lo[...], preferred_element_type=jnp.float32)
            cp = pltpu.make_async_copy(g_vmem, part.at[rows(c), bcols],
                                       gg_sems.at[0])
            cp.start()
            cp.wait()

    def start_rs_step(s):
        rdmas = []
        for r in range(2):
            if r == 0:
                c_send = lax.rem(my - s + N_DEV, N_DEV)
                dst_dev = right
            else:
                c_send = lax.rem(my + s, N_DEV)
                dst_dev = left
            src = part if s == 0 else out_ref
            rdma = pltpu.make_async_remote_copy(
                src_ref=src.at[rows(c_send), cols[r]],
                dst_ref=recv_hbm.at[r, s],
                send_sem=rs_send.at[r, s],
                recv_sem=rs_recv.at[r, s],
                device_id=(dst_dev,),
                device_id_type=pl.DeviceIdType.MESH,
            )
            rdma.start()
            rdmas.append(rdma)
        return rdmas

    def do_adds(s, track_amax, amax):
        for r in range(2):
            if r == 0:
                c_recv = lax.rem(my - s - 1 + N_DEV, N_DEV)
            else:
                c_recv = lax.rem(my + s + 1, N_DEV)
            for t in range(N_SUB):
                tr = pl.ds(t * SUB, SUB)
                trg = pl.ds(c_recv * CHUNK + t * SUB, SUB)
                cp_a = pltpu.make_async_copy(recv_hbm.at[r, s, tr], a_vmem,
                                             copy_sems.at[0])
                cp_b = pltpu.make_async_copy(part.at[trg, cols[r]],
                                             b_vmem, copy_sems.at[1])
                cp_a.start()
                cp_b.start()
                cp_a.wait()
                cp_b.wait()
                a_vmem[...] = a_vmem[...] + b_vmem[...]
                if track_amax:
                    amax = jnp.maximum(amax, jnp.max(a_vmem[...]))
                cp_o = pltpu.make_async_copy(a_vmem, out_ref.at[trg, cols[r]],
                                             copy_sems.at[2])
                cp_o.start()
                cp_o.wait()
        return amax

    def dequant_piece(c, r, scale):
        for t in range(N_SUB):
            trg = pl.ds(c * CHUNK + t * SUB, SUB)
            cp = pltpu.make_async_copy(out8.at[trg, cols[r]], q8_vmem,
                                       copy_sems.at[0])
            cp.start()
            cp.wait()
            a_vmem[...] = q8_vmem[...].astype(jnp.float32) * scale
            cp = pltpu.make_async_copy(a_vmem, out_ref.at[trg, cols[r]],
                                       copy_sems.at[1])
            cp.start()
            cp.wait()

    amax = jnp.float32(0.0)

    gemm_piece(my, 0)
    gemm_piece(my, 1)
    rdmas = start_rs_step(0)
    gemm_piece(lax.rem(my - 1 + N_DEV, N_DEV), 0)
    gemm_piece(lax.rem(my + 1, N_DEV), 1)
    for rdma in rdmas:
        rdma.wait()
    amax = do_adds(0, False, amax)
    rdmas = start_rs_step(1)
    gemm_piece(lax.rem(my + 2, N_DEV), 0)
    gemm_piece(lax.rem(my + 2, N_DEV), 1)
    for rdma in rdmas:
        rdma.wait()
    amax = do_adds(1, False, amax)
    rdmas = start_rs_step(2)
    gemm_piece(lax.rem(my + 1, N_DEV), 0)
    gemm_piece(lax.rem(my - 1 + N_DEV, N_DEV), 1)
    for rdma in rdmas:
        rdma.wait()
    amax = do_adds(2, True, amax)

    mx_comb[...] = jnp.full((8, 128), jnp.maximum(amax, 0.0), jnp.float32)
    for h in range(N_DEV - 1):
        rdma = pltpu.make_async_remote_copy(
            src_ref=mx_comb,
            dst_ref=mx_recv.at[h],
            send_sem=mx_send.at[h],
            recv_sem=mx_recv_sems.at[h],
            device_id=(right,),
            device_id_type=pl.DeviceIdType.MESH,
        )
        rdma.start()
        rdma.wait()
        mx_comb[...] = jnp.maximum(mx_comb[...], mx_recv[h])
    scale = jnp.max(mx_comb[...]) / 448.0

    own = (lax.rem(my + 1, N_DEV), lax.rem(my + N_DEV - 1, N_DEV))
    for r in range(2):
        for t in range(N_SUB):
            trg = pl.ds(own[r] * CHUNK + t * SUB, SUB)
            cp = pltpu.make_async_copy(out_ref.at[trg, cols[r]], a_vmem,
                                       copy_sems.at[0])
            cp.start()
            cp.wait()
            q8_vmem[...] = (jnp.maximum(a_vmem[...], 0.0) / scale).astype(
                jnp.float8_e4m3fn)
            cp = pltpu.make_async_copy(q8_vmem, out8.at[trg, cols[r]],
                                       copy_sems.at[1])
            cp.start()
            cp.wait()

    def start_ag_step(s):
        rdmas = []
        for r in range(2):
            if r == 0:
                c = lax.rem(my + 1 - s + N_DEV, N_DEV)
                dst_dev = right
            else:
                c = lax.rem(my - 1 + s + N_DEV, N_DEV)
                dst_dev = left
            rdma = pltpu.make_async_remote_copy(
                src_ref=out8.at[rows(c), cols[r]],
                dst_ref=out8.at[rows(c), cols[r]],
                send_sem=ag_send.at[r, s],
                recv_sem=ag_recv.at[r, s],
                device_id=(dst_dev,),
                device_id_type=pl.DeviceIdType.MESH,
            )
            rdma.start()
            rdmas.append(rdma)
        return rdmas

    rdmas = start_ag_step(0)
    dequant_piece(own[0], 0, scale)
    dequant_piece(own[1], 1, scale)
    for s in range(1, N_DEV):
        for rdma in rdmas:
            rdma.wait()
        if s < N_DEV - 1:
            rdmas = start_ag_step(s)
        dequant_piece(lax.rem(my - s + 1 + N_DEV, N_DEV), 0, scale)
        dequant_piece(lax.rem(my + s - 1, N_DEV), 1, scale)


def _fused(x, w_mat):
    out, _, _, _ = pl.pallas_call(
        _ar_body,
        out_shape=[
            jax.ShapeDtypeStruct((M, N), jnp.float32),
            jax.ShapeDtypeStruct((M, N), jnp.float8_e4m3fn),
            jax.ShapeDtypeStruct((M, N), jnp.float32),
            jax.ShapeDtypeStruct((2, N_DEV - 1, CHUNK, HALF), jnp.float32),
        ],
        in_specs=[
            pl.BlockSpec(memory_space=pl.ANY),
            pl.BlockSpec(memory_space=pl.ANY),
        ],
        out_specs=[
            pl.BlockSpec(memory_space=pl.ANY),
            pl.BlockSpec(memory_space=pl.ANY),
            pl.BlockSpec(memory_space=pl.ANY),
            pl.BlockSpec(memory_space=pl.ANY),
        ],
        scratch_shapes=[
            pltpu.VMEM((SUB, HALF), jnp.float32),
            pltpu.VMEM((SUB, HALF), jnp.float32),
            pltpu.VMEM((SUB, HALF), jnp.float8_e4m3fn),
            pltpu.VMEM((8, 128), jnp.float32),
            pltpu.VMEM((N_DEV - 1, 8, 128), jnp.float32),
            pltpu.VMEM((CHUNK, KSH), jnp.float32),
            pltpu.VMEM((CHUNK, KSH), jnp.bfloat16),
            pltpu.VMEM((CHUNK, KSH), jnp.bfloat16),
            pltpu.VMEM((KSH, GC), jnp.bfloat16),
            pltpu.VMEM((KSH, GC), jnp.bfloat16),
            pltpu.VMEM((CHUNK, GC), jnp.float32),
            pltpu.SemaphoreType.DMA((2, N_DEV - 1)),
            pltpu.SemaphoreType.DMA((2, N_DEV - 1)),
            pltpu.SemaphoreType.DMA((N_DEV - 1,)),
            pltpu.SemaphoreType.DMA((N_DEV - 1,)),
            pltpu.SemaphoreType.DMA((2, N_DEV - 1)),
            pltpu.SemaphoreType.DMA((2, N_DEV - 1)),
            pltpu.SemaphoreType.DMA((3,)),
            pltpu.SemaphoreType.DMA((1,)),
            pltpu.SemaphoreType.DMA((1,)),
            pltpu.SemaphoreType.DMA((1,)),
        ],
        compiler_params=pltpu.CompilerParams(
            vmem_limit_bytes=60 * 1024 * 1024,
        ),
    )(x, w_mat)
    return out


def kernel(x, w_mat):
    return _fused(x, w_mat)


# device time: 1072221 ns/iter; 2.7006x vs baseline; 1.0003x over previous
import os as _os

import jax

jax.config.update("jax_compilation_cache_dir",
                  _os.path.join(_os.path.dirname(__file__), ".jax_cache"))
jax.config.update("jax_persistent_cache_min_compile_time_secs", 0)
jax.config.update("jax_persistent_cache_min_entry_size_bytes", 0)

import jax.numpy as jnp
from jax import lax
from jax.experimental import pallas as pl
from jax.experimental.pallas import tpu as pltpu

N_DEV = 4
M, K, N = 4096, 4096, 8192
KSH = K // N_DEV
CHUNK = M // N_DEV
HALF = N // 2
SUB = 256
N_SUB = CHUNK // SUB
GR = 512
GC = 2048


def _ar_body(x_ref, w_ref, out_ref, out8, part, recv_hbm,
             a_vmem, b_vmem, q8_vmem, mx_comb, mx_recv,
             xf_vmem, x_hi, x_lo, w_hi, w_lo, g_vmem,
             rs_send, rs_recv, mx_send, mx_recv_sems, ag_send, ag_recv,
             copy_sems, gx_sems, gw_sems, gg_sems):
    my = lax.axis_index("i")
    right = lax.rem(my + 1, N_DEV)
    left = lax.rem(my + N_DEV - 1, N_DEV)

    def rows(c):
        return pl.ds(c * CHUNK, CHUNK)

    cols = (pl.ds(0, HALF), pl.ds(HALF, HALF))

    def gemm_piece(c, r):
        base = r * HALF
        cp = pltpu.make_async_copy(x_ref.at[rows(c)], xf_vmem, gx_sems.at[0])
        cp.start()
        cp.wait()
        x_hi[...] = xf_vmem[...].astype(jnp.bfloat16)
        x_lo[...] = (xf_vmem[...] - x_hi[...].astype(jnp.float32)).astype(
            jnp.bfloat16)
        for j in range(2):
            bcols = pl.ds(base + j * GC, GC)
            cp = pltpu.make_async_copy(w_ref.at[:, bcols], g_vmem,
                                       gw_sems.at[0])
            cp.start()
            cp.wait()
            w_hi[...] = g_vmem[...].astype(jnp.bfloat16)
            w_lo[...] = (g_vmem[...] - w_hi[...].astype(jnp.float32)).astype(
                jnp.bfloat16)
            g_vmem[...] = jnp.dot(x_hi[...], w_hi[...],
                                  preferred_element_type=jnp.float32)
            g_vmem[...] = g_vmem[...] + jnp.dot(
                x_lo[...], w_hi[...], preferred_element_type=jnp.float32)
            g_vmem[...] = g_vmem[...] + jnp.dot(
                x_hi[...], w_lo[...], preferred_element_type=jnp.float32)
            cp = pltpu.make_async_copy(g_vmem, part.at[rows(c), bcols],
                                       gg_sems.at[0])
            cp.start()
            cp.wait()

    def start_rs_step(s):
        rdmas = []
        for r in range(2):
            if r == 0:
                c_send = lax.rem(my - s + N_DEV, N_DEV)
                dst_dev = right
            else:
                c_send = lax.rem(my + s, N_DEV)
                dst_dev = left
            src = part if s == 0 else out_ref
            rdma = pltpu.make_async_remote_copy(
                src_ref=src.at[rows(c_send), cols[r]],
                dst_ref=recv_hbm.at[r, s],
                send_sem=rs_send.at[r, s],
                recv_sem=rs_recv.at[r, s],
                device_id=(dst_dev,),
                device_id_type=pl.DeviceIdType.MESH,
            )
            rdma.start()
            rdmas.append(rdma)
        return rdmas

    def do_adds(s, track_amax, amax):
        for r in range(2):
            if r == 0:
                c_recv = lax.rem(my - s - 1 + N_DEV, N_DEV)
            else:
                c_recv = lax.rem(my + s + 1, N_DEV)
            for t in range(N_SUB):
                tr = pl.ds(t * SUB, SUB)
                trg = pl.ds(c_recv * CHUNK + t * SUB, SUB)
                cp_a = pltpu.make_async_copy(recv_hbm.at[r, s, tr], a_vmem,
                                             copy_sems.at[0])
                cp_b = pltpu.make_async_copy(part.at[trg, cols[r]],
                                             b_vmem, copy_sems.at[1])
                cp_a.start()
                cp_b.start()
                cp_a.wait()
                cp_b.wait()
                a_vmem[...] = a_vmem[...] + b_vmem[...]
                if track_amax:
                    amax = jnp.maximum(amax, jnp.max(a_vmem[...]))
                cp_o = pltpu.make_async_copy(a_vmem, out_ref.at[trg, cols[r]],
                                             copy_sems.at[2])
                cp_o.start()
                cp_o.wait()
        return amax

    def dequant_piece(c, r, scale):
        for t in range(N_SUB):
            trg = pl.ds(c * CHUNK + t * SUB, SUB)
            cp = pltpu.make_async_copy(out8.at[trg, cols[r]], q8_vmem,
                                       copy_sems.at[0])
            cp.start()
            cp.wait()
            a_vmem[...] = q8_vmem[...].astype(jnp.float32) * scale
            cp = pltpu.make_async_copy(a_vmem, out_ref.at[trg, cols[r]],
                                       copy_sems.at[1])
            cp.start()
            cp.wait()

    amax = jnp.float32(0.0)

    gemm_piece(my, 0)
    gemm_piece(my, 1)
    rdmas = start_rs_step(0)
    gemm_piece(lax.rem(my - 1 + N_DEV, N_DEV), 0)
    gemm_piece(lax.rem(my + 1, N_DEV), 1)
    for rdma in rdmas:
        rdma.wait()
    amax = do_adds(0, False, amax)
    rdmas = start_rs_step(1)
    gemm_piece(lax.rem(my + 2, N_DEV), 0)
    gemm_piece(lax.rem(my + 2, N_DEV), 1)
    for rdma in rdmas:
        rdma.wait()
    amax = do_adds(1, False, amax)
    rdmas = start_rs_step(2)
    gemm_piece(lax.rem(my + 1, N_DEV), 0)
    gemm_piece(lax.rem(my - 1 + N_DEV, N_DEV), 1)
    for rdma in rdmas:
        rdma.wait()
    amax = do_adds(2, True, amax)

    mx_comb[...] = jnp.full((8, 128), jnp.maximum(amax, 0.0), jnp.float32)
    for h in range(N_DEV - 1):
        rdma = pltpu.make_async_remote_copy(
            src_ref=mx_comb,
            dst_ref=mx_recv.at[h],
            send_sem=mx_send.at[h],
            recv_sem=mx_recv_sems.at[h],
            device_id=(right,),
            device_id_type=pl.DeviceIdType.MESH,
        )
        rdma.start()
        rdma.wait()
        mx_comb[...] = jnp.maximum(mx_comb[...], mx_recv[h])
    scale = jnp.max(mx_comb[...]) / 448.0

    own = (lax.rem(my + 1, N_DEV), lax.rem(my + N_DEV - 1, N_DEV))
    for r in range(2):
        for t in range(N_SUB):
            trg = pl.ds(own[r] * CHUNK + t * SUB, SUB)
            cp = pltpu.make_async_copy(out_ref.at[trg, cols[r]], a_vmem,
                                       copy_sems.at[0])
            cp.start()
            cp.wait()
            q8_vmem[...] = (jnp.maximum(a_vmem[...], 0.0) / scale).astype(
                jnp.float8_e4m3fn)
            cp = pltpu.make_async_copy(q8_vmem, out8.at[trg, cols[r]],
                                       copy_sems.at[1])
            cp.start()
            cp.wait()

    def start_ag_step(s):
        rdmas = []
        for r in range(2):
            if r == 0:
                c = lax.rem(my + 1 - s + N_DEV, N_DEV)
                dst_dev = right
            else:
                c = lax.rem(my - 1 + s + N_DEV, N_DEV)
                dst_dev = left
            rdma = pltpu.make_async_remote_copy(
                src_ref=out8.at[rows(c), cols[r]],
                dst_ref=out8.at[rows(c), cols[r]],
                send_sem=ag_send.at[r, s],
                recv_sem=ag_recv.at[r, s],
                device_id=(dst_dev,),
                device_id_type=pl.DeviceIdType.MESH,
            )
            rdma.start()
            rdmas.append(rdma)
        return rdmas

    rdmas = start_ag_step(0)
    dequant_piece(own[0], 0, scale)
    dequant_piece(own[1], 1, scale)
    for s in range(1, N_DEV):
        for rdma in rdmas:
            rdma.wait()
        if s < N_DEV - 1:
            rdmas = start_ag_step(s)
        dequant_piece(lax.rem(my - s + 1 + N_DEV, N_DEV), 0, scale)
        dequant_piece(lax.rem(my + s - 1, N_DEV), 1, scale)


def _fused(x, w_mat):
    out, _, _, _ = pl.pallas_call(
        _ar_body,
        out_shape=[
            jax.ShapeDtypeStruct((M, N), jnp.float32),
            jax.ShapeDtypeStruct((M, N), jnp.float8_e4m3fn),
            jax.ShapeDtypeStruct((M, N), jnp.float32),
            jax.ShapeDtypeStruct((2, N_DEV - 1, CHUNK, HALF), jnp.float32),
        ],
        in_specs=[
            pl.BlockSpec(memory_space=pl.ANY),
            pl.BlockSpec(memory_space=pl.ANY),
        ],
        out_specs=[
            pl.BlockSpec(memory_space=pl.ANY),
            pl.BlockSpec(memory_space=pl.ANY),
            pl.BlockSpec(memory_space=pl.ANY),
            pl.BlockSpec(memory_space=pl.ANY),
        ],
        scratch_shapes=[
            pltpu.VMEM((SUB, HALF), jnp.float32),
            pltpu.VMEM((SUB, HALF), jnp.float32),
            pltpu.VMEM((SUB, HALF), jnp.float8_e4m3fn),
            pltpu.VMEM((8, 128), jnp.float32),
            pltpu.VMEM((N_DEV - 1, 8, 128), jnp.float32),
            pltpu.VMEM((CHUNK, KSH), jnp.float32),
            pltpu.VMEM((CHUNK, KSH), jnp.bfloat16),
            pltpu.VMEM((CHUNK, KSH), jnp.bfloat16),
            pltpu.VMEM((KSH, GC), jnp.bfloat16),
            pltpu.VMEM((KSH, GC), jnp.bfloat16),
            pltpu.VMEM((CHUNK, GC), jnp.float32),
            pltpu.SemaphoreType.DMA((2, N_DEV - 1)),
            pltpu.SemaphoreType.DMA((2, N_DEV - 1)),
            pltpu.SemaphoreType.DMA((N_DEV - 1,)),
            pltpu.SemaphoreType.DMA((N_DEV - 1,)),
            pltpu.SemaphoreType.DMA((2, N_DEV - 1)),
            pltpu.SemaphoreType.DMA((2, N_DEV - 1)),
            pltpu.SemaphoreType.DMA((3,)),
            pltpu.SemaphoreType.DMA((1,)),
            pltpu.SemaphoreType.DMA((1,)),
            pltpu.SemaphoreType.DMA((1,)),
        ],
        compiler_params=pltpu.CompilerParams(
            vmem_limit_bytes=60 * 1024 * 1024,
        ),
    )(x, w_mat)
    return out


def kernel(x, w_mat):
    return _fused(x, w_mat)


def _warm_cache():
    try:
        import distributed_mesh_v7x as dm
        from jax.experimental.shard_map import shard_map
        from jax.sharding import NamedSharding, PartitionSpec as P

        mesh = dm.get_mesh("i", world_size=4)
        xp, wp = P(None, "i"), P("i", None)
        xs = jax.ShapeDtypeStruct((M, K), jnp.float32,
                                  sharding=NamedSharding(mesh, xp))
        ws = jax.ShapeDtypeStruct((K, N), jnp.float32,
                                  sharding=NamedSharding(mesh, wp))
        jax.jit(
            shard_map(kernel, mesh=mesh, in_specs=(xp, wp),
                      out_specs=P(None, None), check_rep=False)
        ).lower(xs, ws).compile()
    except Exception:
        pass


_warm_cache()
